# baseline (speedup 1.0000x reference)
import functools

import jax
import jax.numpy as jnp
from jax.experimental import pallas as pl
from jax.experimental.pallas import tpu as pltpu

_K_PERCENT = (10, 10)


def _bce_body(p_ref, t_ref, out_ref):
    p = p_ref[...]
    t = t_ref[...]
    log_p = jnp.maximum(jnp.log(p), -100.0)
    log_1mp = jnp.maximum(jnp.log(1.0 - p), -100.0)
    out_ref[...] = -(t * log_p + (1.0 - t) * log_1mp)


def _bce_loss_channel_major(predict, target):
    B, C, H, W = predict.shape
    grid = (C, B)
    return pl.pallas_call(
        _bce_body,
        grid=grid,
        in_specs=[
            pl.BlockSpec((1, 1, H, W), lambda c, b: (b, c, 0, 0)),
            pl.BlockSpec((1, 1, H, W), lambda c, b: (b, c, 0, 0)),
        ],
        out_specs=pl.BlockSpec((1, 1, H, W), lambda c, b: (c, b, 0, 0)),
        out_shape=jax.ShapeDtypeStruct((C, B, H, W), jnp.float32),
    )(predict, target)


def kernel(predict, target, is_average):
    B, C, H, W = predict.shape
    n = B * H * W
    loss = _bce_loss_channel_major(predict, target).reshape(C, n)
    total = jnp.float32(0.0)
    for c in range(C):
        kk = int(n * _K_PERCENT[c] / 100)
        topv, _ = jax.lax.top_k(loss[c], kk)
        total = total + topv.mean()
    total = total / C
    return jnp.where(is_average, total, total * B)


# speedup vs baseline: 11.3481x; 11.3481x over previous
import functools

import jax
import jax.numpy as jnp
from jax import lax
from jax.experimental import pallas as pl
from jax.experimental.pallas import tpu as pltpu
from jax.experimental.pallas import tpu_sc as plsc

_K_PERCENT = (10, 10)

_B, _C, _H, _W = 8, 2, 512, 512
_N = _B * _H * _W
_KK = int(_N * _K_PERCENT[0] / 100)
_NS = 16
_NT = _N // _NS
_CH = 16384
_NCHUNK = _NT // _CH
_BINS = 256
_L = 16


def _bce_body(p_ref, t_ref, out_ref):
    p = p_ref[...]
    t = t_ref[...]
    log_p = jnp.maximum(jnp.log(p), -100.0)
    log_1mp = jnp.maximum(jnp.log(1.0 - p), -100.0)
    out_ref[...] = -(t * log_p + (1.0 - t) * log_1mp)


def _bce_loss_channel_major(predict, target):
    return pl.pallas_call(
        _bce_body,
        grid=(_C, _B),
        in_specs=[
            pl.BlockSpec((1, 1, _H, _W), lambda c, b: (b, c, 0, 0)),
            pl.BlockSpec((1, 1, _H, _W), lambda c, b: (b, c, 0, 0)),
        ],
        out_specs=pl.BlockSpec((1, 1, _H, _W), lambda c, b: (c, b, 0, 0)),
        out_shape=jax.ShapeDtypeStruct((_C, _B, _H, _W), jnp.float32),
    )(predict, target)


def _sc_body(loss_ref, out_ref, buf, cnt, sm, mbuf, macc, stage, mc, ms,
             sh_cnt, sh_sum, sh_mc, sh_ms):
    ch = lax.axis_index("c")
    s = lax.axis_index("s")
    lane = lax.iota(jnp.int32, _L)
    ones = jnp.ones((_L,), jnp.float32)
    zeros16 = jnp.zeros((_L,), jnp.float32)

    rem = jnp.full((_L,), float(_KK), jnp.float32)
    s_inc = jnp.zeros((_L,), jnp.float32)
    pref = jnp.zeros((_L,), jnp.int32)

    for shift in (24, 16, 8, 0):
        def zero_body(q, _):
            cnt[pl.ds(q * _L, _L)] = zeros16
            sm[pl.ds(q * _L, _L)] = zeros16
            return 0
        lax.fori_loop(0, _BINS, zero_body, 0)

        pref_now = pref

        def chunk_body(j, _):
            pltpu.sync_copy(
                loss_ref.at[ch, pl.ds(s * _NT + j * _CH, _CH)], buf)

            def it_body(i, _):
                v = buf[pl.ds(i * _L, _L)]
                bits = lax.bitcast_convert_type(v, jnp.int32)
                b = lax.shift_right_logical(bits, shift) & 255
                idx = b * _L + lane
                if shift == 24:
                    plsc.addupdate_scatter(cnt, [idx], ones)
                    plsc.addupdate_scatter(sm, [idx], v)
                else:
                    m = lax.shift_right_logical(bits, shift + 8) == pref_now
                    plsc.addupdate_scatter(cnt, [idx], ones, mask=m)
                    plsc.addupdate_scatter(sm, [idx], v, mask=m)
                return 0
            lax.fori_loop(0, _CH // _L, it_body, 0)
            return 0
        lax.fori_loop(0, _NCHUNK, chunk_body, 0)

        pltpu.sync_copy(cnt, sh_cnt.at[s])
        pltpu.sync_copy(sm, sh_sum.at[s])
        plsc.subcore_barrier()

        for src, dst in ((sh_cnt, sh_mc), (sh_sum, sh_ms)):
            pltpu.sync_copy(src.at[:, pl.ds(s * _BINS, _BINS)], mbuf)

            def zacc(q, _):
                macc[pl.ds(q * _L, _L)] = zeros16
                return 0
            lax.fori_loop(0, _L, zacc, 0)

            def row_body(r, _):
                def q_body(q, _):
                    macc[pl.ds(q * _L, _L)] = (
                        macc[pl.ds(q * _L, _L)] + mbuf[r, pl.ds(q * _L, _L)])
                    return 0
                lax.fori_loop(0, _L, q_body, 0)
                return 0
            lax.fori_loop(0, _NS, row_body, 0)

            def lred(k, tot):
                return jnp.where(lane == k,
                                 jnp.sum(macc[pl.ds(k * _L, _L)]), tot)
            totv = lax.fori_loop(0, _L, lred, zeros16)
            stage[...] = totv
            pltpu.sync_copy(stage, dst.at[pl.ds(s * _L, _L)])
        plsc.subcore_barrier()

        pltpu.sync_copy(sh_mc, mc)
        pltpu.sync_copy(sh_ms, ms)

        def scan_body(gi, carry):
            c0, s1, rem1, binv, found = carry
            g = 15 - gi
            cv = mc[pl.ds(g * _L, _L)]
            sv = ms[pl.ds(g * _L, _L)]
            rv = lax.rev(cv, (0,))
            rs = lax.rev(sv, (0,))
            cum = plsc.cumsum(rv)
            cums = plsc.cumsum(rs)
            gtot = jnp.sum(cv)
            gstot = jnp.sum(sv)
            any_hit = (c0 + gtot) >= rem1
            hit = jnp.logical_and(jnp.logical_not(found),
                                  (c0 + cum) >= rem1)
            i = plsc.all_reduce_ffs(hit)
            sel = lane == i
            rv_i = jnp.sum(jnp.where(sel, rv, 0.0))
            cum_i = jnp.sum(jnp.where(sel, cum, 0.0))
            cums_i = jnp.sum(jnp.where(sel, cums, 0.0))
            rs_i = jnp.sum(jnp.where(sel, rs, 0.0))
            in_group = jnp.logical_and(jnp.logical_not(found), any_hit)
            take_all = jnp.logical_and(jnp.logical_not(found),
                                       jnp.logical_not(any_hit))
            s2 = jnp.where(in_group, s1 + (cums_i - rs_i),
                           jnp.where(take_all, s1 + gstot, s1))
            rem2 = jnp.where(in_group, rem1 - (c0 + cum_i - rv_i), rem1)
            binv2 = jnp.where(in_group, g * _L + (15 - i), binv)
            found2 = jnp.logical_or(found, in_group)
            return (c0 + gtot, s2, rem2, binv2, found2)

        init = (jnp.zeros((_L,), jnp.float32), s_inc, rem,
                jnp.zeros((_L,), jnp.int32), jnp.zeros((_L,), jnp.bool_))
        _, s_inc, rem, binstar, _ = lax.fori_loop(0, _L, scan_body, init)
        pref = lax.shift_left(pref, 8) | binstar

    tval = lax.bitcast_convert_type(pref, jnp.float32)
    res = s_inc + rem * tval

    @pl.when(s == 0)
    def _():
        stage[...] = res
        pltpu.sync_copy(stage, out_ref.at[ch])


def _sc_topk_sums(loss):
    mesh = plsc.VectorSubcoreMesh(core_axis_name="c", subcore_axis_name="s")
    f = pl.kernel(
        _sc_body,
        out_type=jax.ShapeDtypeStruct((_C, _L), jnp.float32),
        mesh=mesh,
        compiler_params=pltpu.CompilerParams(needs_layout_passes=False),
        scratch_types=[
            pltpu.VMEM((_CH,), jnp.float32),
            pltpu.VMEM((_BINS * _L,), jnp.float32),
            pltpu.VMEM((_BINS * _L,), jnp.float32),
            pltpu.VMEM((_NS, _BINS), jnp.float32),
            pltpu.VMEM((_BINS,), jnp.float32),
            pltpu.VMEM((_L,), jnp.float32),
            pltpu.VMEM((_BINS,), jnp.float32),
            pltpu.VMEM((_BINS,), jnp.float32),
            pltpu.VMEM_SHARED((_NS, _BINS * _L), jnp.float32),
            pltpu.VMEM_SHARED((_NS, _BINS * _L), jnp.float32),
            pltpu.VMEM_SHARED((_BINS,), jnp.float32),
            pltpu.VMEM_SHARED((_BINS,), jnp.float32),
        ],
    )
    return f(loss)


def kernel(predict, target, is_average):
    loss = _bce_loss_channel_major(predict, target).reshape(_C, _N)
    sums = _sc_topk_sums(loss)
    total = (sums[0, 0] + sums[1, 0]) / (_KK * _C)
    return jnp.where(is_average, total, total * _B)


# speedup vs baseline: 23.8304x; 2.1000x over previous
import functools

import jax
import jax.numpy as jnp
from jax import lax
from jax.experimental import pallas as pl
from jax.experimental.pallas import tpu as pltpu
from jax.experimental.pallas import tpu_sc as plsc

_K_PERCENT = (10, 10)

_B, _C, _H, _W = 8, 2, 512, 512
_N = _B * _H * _W
_KK = int(_N * _K_PERCENT[0] / 100)
_NS = 16
_NT = _N // _NS
_CH = 16384
_NCHUNK = _NT // _CH
_BINS = 256
_L = 16


def _bce_body(p_ref, t_ref, out_ref):
    p = p_ref[...]
    t = t_ref[...]
    log_p = jnp.maximum(jnp.log(p), -100.0)
    log_1mp = jnp.maximum(jnp.log(1.0 - p), -100.0)
    out_ref[...] = -(t * log_p + (1.0 - t) * log_1mp)


def _bce_loss_channel_major(predict, target):
    return pl.pallas_call(
        _bce_body,
        grid=(_C, _B),
        in_specs=[
            pl.BlockSpec((1, 1, _H, _W), lambda c, b: (b, c, 0, 0)),
            pl.BlockSpec((1, 1, _H, _W), lambda c, b: (b, c, 0, 0)),
        ],
        out_specs=pl.BlockSpec((1, 1, _H, _W), lambda c, b: (c, b, 0, 0)),
        out_shape=jax.ShapeDtypeStruct((_C, _B, _H, _W), jnp.float32),
    )(predict, target)


def _sc_body(loss_ref, out_ref, buf, cnt, sm, mbuf, macc, stage, mc, ms,
             sh_cnt, sh_sum, sh_mc, sh_ms):
    ch = lax.axis_index("c")
    s = lax.axis_index("s")
    lane = lax.iota(jnp.int32, _L)
    ones = jnp.ones((_L,), jnp.float32)
    zeros16 = jnp.zeros((_L,), jnp.float32)

    rem = jnp.full((_L,), float(_KK), jnp.float32)
    s_inc = jnp.zeros((_L,), jnp.float32)
    pref = jnp.zeros((_L,), jnp.int32)

    for shift in (24, 16, 8, 0):
        def zero_body(q, _):
            cnt[pl.ds(q * _L, _L)] = zeros16
            sm[pl.ds(q * _L, _L)] = zeros16
            return 0
        lax.fori_loop(0, _BINS, zero_body, 0)

        pref_now = pref

        def chunk_body(j, _):
            pltpu.sync_copy(
                loss_ref.at[ch, pl.ds(s * _NT + j * _CH, _CH)], buf)

            @plsc.parallel_loop(0, _CH // _L, 1, unroll=8)
            def it_body(i):
                v = buf[pl.ds(i * _L, _L)]
                bits = lax.bitcast_convert_type(v, jnp.int32)
                b = lax.shift_right_logical(bits, shift) & 255
                idx = b * _L + lane
                if shift == 24:
                    plsc.addupdate_scatter(cnt, [idx], ones)
                    plsc.addupdate_scatter(sm, [idx], v)
                else:
                    m = lax.shift_right_logical(bits, shift + 8) == pref_now
                    plsc.addupdate_scatter(cnt, [idx], ones, mask=m)
                    plsc.addupdate_scatter(sm, [idx], v, mask=m)
            return 0
        lax.fori_loop(0, _NCHUNK, chunk_body, 0)

        pltpu.sync_copy(cnt, sh_cnt.at[s])
        pltpu.sync_copy(sm, sh_sum.at[s])
        plsc.subcore_barrier()

        for src, dst in ((sh_cnt, sh_mc), (sh_sum, sh_ms)):
            pltpu.sync_copy(src.at[:, pl.ds(s * _BINS, _BINS)], mbuf)

            def zacc(q, _):
                macc[pl.ds(q * _L, _L)] = zeros16
                return 0
            lax.fori_loop(0, _L, zacc, 0)

            def row_body(r, _):
                def q_body(q, _):
                    macc[pl.ds(q * _L, _L)] = (
                        macc[pl.ds(q * _L, _L)] + mbuf[r, pl.ds(q * _L, _L)])
                    return 0
                lax.fori_loop(0, _L, q_body, 0)
                return 0
            lax.fori_loop(0, _NS, row_body, 0)

            def lred(k, tot):
                return jnp.where(lane == k,
                                 jnp.sum(macc[pl.ds(k * _L, _L)]), tot)
            totv = lax.fori_loop(0, _L, lred, zeros16)
            stage[...] = totv
            pltpu.sync_copy(stage, dst.at[pl.ds(s * _L, _L)])
        plsc.subcore_barrier()

        pltpu.sync_copy(sh_mc, mc)
        pltpu.sync_copy(sh_ms, ms)

        def scan_body(gi, carry):
            c0, s1, rem1, binv, found = carry
            g = 15 - gi
            cv = mc[pl.ds(g * _L, _L)]
            sv = ms[pl.ds(g * _L, _L)]
            rv = lax.rev(cv, (0,))
            rs = lax.rev(sv, (0,))
            cum = plsc.cumsum(rv)
            cums = plsc.cumsum(rs)
            gtot = jnp.sum(cv)
            gstot = jnp.sum(sv)
            any_hit = (c0 + gtot) >= rem1
            hit = jnp.logical_and(jnp.logical_not(found),
                                  (c0 + cum) >= rem1)
            i = plsc.all_reduce_ffs(hit)
            sel = lane == i
            rv_i = jnp.sum(jnp.where(sel, rv, 0.0))
            cum_i = jnp.sum(jnp.where(sel, cum, 0.0))
            cums_i = jnp.sum(jnp.where(sel, cums, 0.0))
            rs_i = jnp.sum(jnp.where(sel, rs, 0.0))
            in_group = jnp.logical_and(jnp.logical_not(found), any_hit)
            take_all = jnp.logical_and(jnp.logical_not(found),
                                       jnp.logical_not(any_hit))
            s2 = jnp.where(in_group, s1 + (cums_i - rs_i),
                           jnp.where(take_all, s1 + gstot, s1))
            rem2 = jnp.where(in_group, rem1 - (c0 + cum_i - rv_i), rem1)
            binv2 = jnp.where(in_group, g * _L + (15 - i), binv)
            found2 = jnp.logical_or(found, in_group)
            return (c0 + gtot, s2, rem2, binv2, found2)

        init = (jnp.zeros((_L,), jnp.float32), s_inc, rem,
                jnp.zeros((_L,), jnp.int32), jnp.zeros((_L,), jnp.bool_))
        _, s_inc, rem, binstar, _ = lax.fori_loop(0, _L, scan_body, init)
        pref = lax.shift_left(pref, 8) | binstar

    tval = lax.bitcast_convert_type(pref, jnp.float32)
    res = s_inc + rem * tval

    @pl.when(s == 0)
    def _():
        stage[...] = res
        pltpu.sync_copy(stage, out_ref.at[ch])


def _sc_topk_sums(loss):
    mesh = plsc.VectorSubcoreMesh(core_axis_name="c", subcore_axis_name="s")
    f = pl.kernel(
        _sc_body,
        out_type=jax.ShapeDtypeStruct((_C, _L), jnp.float32),
        mesh=mesh,
        compiler_params=pltpu.CompilerParams(needs_layout_passes=False),
        scratch_types=[
            pltpu.VMEM((_CH,), jnp.float32),
            pltpu.VMEM((_BINS * _L,), jnp.float32),
            pltpu.VMEM((_BINS * _L,), jnp.float32),
            pltpu.VMEM((_NS, _BINS), jnp.float32),
            pltpu.VMEM((_BINS,), jnp.float32),
            pltpu.VMEM((_L,), jnp.float32),
            pltpu.VMEM((_BINS,), jnp.float32),
            pltpu.VMEM((_BINS,), jnp.float32),
            pltpu.VMEM_SHARED((_NS, _BINS * _L), jnp.float32),
            pltpu.VMEM_SHARED((_NS, _BINS * _L), jnp.float32),
            pltpu.VMEM_SHARED((_BINS,), jnp.float32),
            pltpu.VMEM_SHARED((_BINS,), jnp.float32),
        ],
    )
    return f(loss)


def kernel(predict, target, is_average):
    loss = _bce_loss_channel_major(predict, target).reshape(_C, _N)
    sums = _sc_topk_sums(loss)
    total = (sums[0, 0] + sums[1, 0]) / (_KK * _C)
    return jnp.where(is_average, total, total * _B)


# speedup vs baseline: 27.1413x; 1.1389x over previous
import functools

import jax
import jax.numpy as jnp
from jax import lax
from jax.experimental import pallas as pl
from jax.experimental.pallas import tpu as pltpu
from jax.experimental.pallas import tpu_sc as plsc

_K_PERCENT = (10, 10)

_B, _C, _H, _W = 8, 2, 512, 512
_N = _B * _H * _W
_KK = int(_N * _K_PERCENT[0] / 100)
_NS = 16
_NT = _N // _NS
_CH = 16384
_NCHUNK = _NT // _CH
_BINS = 256
_L = 16


def _bce_body(p_ref, t_ref, out_ref):
    p = p_ref[...]
    t = t_ref[...]
    log_p = jnp.maximum(jnp.log(p), -100.0)
    log_1mp = jnp.maximum(jnp.log(1.0 - p), -100.0)
    out_ref[...] = -(t * log_p + (1.0 - t) * log_1mp)


def _bce_loss_channel_major(predict, target):
    return pl.pallas_call(
        _bce_body,
        grid=(_C, _B),
        in_specs=[
            pl.BlockSpec((1, 1, _H, _W), lambda c, b: (b, c, 0, 0)),
            pl.BlockSpec((1, 1, _H, _W), lambda c, b: (b, c, 0, 0)),
        ],
        out_specs=pl.BlockSpec((1, 1, _H, _W), lambda c, b: (c, b, 0, 0)),
        out_shape=jax.ShapeDtypeStruct((_C, _B, _H, _W), jnp.float32),
    )(predict, target)


def _sc_body(loss_ref, out_ref, buf, cnt, sm, mbuf, macc, stage, mc, ms,
             sema, semb, sh_cnt, sh_sum, sh_mc, sh_ms):
    ch = lax.axis_index("c")
    s = lax.axis_index("s")
    lane = lax.iota(jnp.int32, _L)
    ones = jnp.ones((_L,), jnp.float32)
    zeros16 = jnp.zeros((_L,), jnp.float32)

    rem = jnp.full((_L,), float(_KK), jnp.float32)
    s_inc = jnp.zeros((_L,), jnp.float32)
    pref = jnp.zeros((_L,), jnp.int32)

    def chunk_src(j):
        return loss_ref.at[ch, pl.ds(s * _NT + j * _CH, _CH)]

    for shift in (24, 16, 8, 0):
        @plsc.parallel_loop(0, _BINS, 1, unroll=8)
        def zero_body(q):
            cnt[pl.ds(q * _L, _L)] = zeros16
            sm[pl.ds(q * _L, _L)] = zeros16

        pref_now = pref

        def process(p):
            @plsc.parallel_loop(0, _CH // _L, 1, unroll=8)
            def it_body(i):
                v = buf[p, pl.ds(i * _L, _L)]
                bits = lax.bitcast_convert_type(v, jnp.int32)
                b = lax.shift_right_logical(bits, shift) & 255
                idx = b * _L + lane
                if shift == 24:
                    plsc.addupdate_scatter(cnt, [idx], ones)
                    plsc.addupdate_scatter(sm, [idx], v)
                else:
                    m = lax.shift_right_logical(bits, shift + 8) == pref_now
                    plsc.addupdate_scatter(cnt, [idx], ones, mask=m)
                    plsc.addupdate_scatter(sm, [idx], v, mask=m)

        pltpu.async_copy(chunk_src(0), buf.at[0], sema)
        pltpu.async_copy(chunk_src(1), buf.at[1], semb)

        def dchunk_body(m, _):
            j0 = 2 * m
            pltpu.make_async_copy(chunk_src(j0), buf.at[0], sema).wait()
            process(0)
            pltpu.async_copy(
                chunk_src(jnp.minimum(j0 + 2, _NCHUNK - 2)), buf.at[0], sema)
            pltpu.make_async_copy(chunk_src(j0 + 1), buf.at[1], semb).wait()
            process(1)
            pltpu.async_copy(
                chunk_src(jnp.minimum(j0 + 3, _NCHUNK - 1)), buf.at[1], semb)
            return 0
        lax.fori_loop(0, _NCHUNK // 2, dchunk_body, 0)
        pltpu.make_async_copy(chunk_src(_NCHUNK - 2), buf.at[0], sema).wait()
        pltpu.make_async_copy(chunk_src(_NCHUNK - 1), buf.at[1], semb).wait()

        pltpu.sync_copy(cnt, sh_cnt.at[s])
        pltpu.sync_copy(sm, sh_sum.at[s])
        plsc.subcore_barrier()

        for src, dst in ((sh_cnt, sh_mc), (sh_sum, sh_ms)):
            pltpu.sync_copy(src.at[:, pl.ds(s * _BINS, _BINS)], mbuf)

            @plsc.parallel_loop(0, _L, 1, unroll=4)
            def zacc(q):
                macc[pl.ds(q * _L, _L)] = zeros16

            def row_body(r, _):
                @plsc.parallel_loop(0, _L, 1, unroll=4)
                def q_body(q):
                    macc[pl.ds(q * _L, _L)] = (
                        macc[pl.ds(q * _L, _L)] + mbuf[r, pl.ds(q * _L, _L)])
                return 0
            lax.fori_loop(0, _NS, row_body, 0)

            def lred(k, tot):
                return jnp.where(lane == k,
                                 jnp.sum(macc[pl.ds(k * _L, _L)]), tot)
            totv = lax.fori_loop(0, _L, lred, zeros16)
            stage[...] = totv
            pltpu.sync_copy(stage, dst.at[pl.ds(s * _L, _L)])
        plsc.subcore_barrier()

        pltpu.sync_copy(sh_mc, mc)
        pltpu.sync_copy(sh_ms, ms)

        def scan_body(gi, carry):
            c0, s1, rem1, binv, found = carry
            g = 15 - gi
            cv = mc[pl.ds(g * _L, _L)]
            sv = ms[pl.ds(g * _L, _L)]
            rv = lax.rev(cv, (0,))
            rs = lax.rev(sv, (0,))
            cum = plsc.cumsum(rv)
            cums = plsc.cumsum(rs)
            gtot = jnp.sum(cv)
            gstot = jnp.sum(sv)
            any_hit = (c0 + gtot) >= rem1
            hit = jnp.logical_and(jnp.logical_not(found),
                                  (c0 + cum) >= rem1)
            i = plsc.all_reduce_ffs(hit)
            sel = lane == i
            rv_i = jnp.sum(jnp.where(sel, rv, 0.0))
            cum_i = jnp.sum(jnp.where(sel, cum, 0.0))
            cums_i = jnp.sum(jnp.where(sel, cums, 0.0))
            rs_i = jnp.sum(jnp.where(sel, rs, 0.0))
            in_group = jnp.logical_and(jnp.logical_not(found), any_hit)
            take_all = jnp.logical_and(jnp.logical_not(found),
                                       jnp.logical_not(any_hit))
            s2 = jnp.where(in_group, s1 + (cums_i - rs_i),
                           jnp.where(take_all, s1 + gstot, s1))
            rem2 = jnp.where(in_group, rem1 - (c0 + cum_i - rv_i), rem1)
            binv2 = jnp.where(in_group, g * _L + (15 - i), binv)
            found2 = jnp.logical_or(found, in_group)
            return (c0 + gtot, s2, rem2, binv2, found2)

        init = (jnp.zeros((_L,), jnp.float32), s_inc, rem,
                jnp.zeros((_L,), jnp.int32), jnp.zeros((_L,), jnp.bool_))
        _, s_inc, rem, binstar, _ = lax.fori_loop(0, _L, scan_body, init)
        pref = lax.shift_left(pref, 8) | binstar

    tval = lax.bitcast_convert_type(pref, jnp.float32)
    res = s_inc + rem * tval

    @pl.when(s == 0)
    def _():
        stage[...] = res
        pltpu.sync_copy(stage, out_ref.at[ch])


def _sc_topk_sums(loss):
    mesh = plsc.VectorSubcoreMesh(core_axis_name="c", subcore_axis_name="s")
    f = pl.kernel(
        _sc_body,
        out_type=jax.ShapeDtypeStruct((_C, _L), jnp.float32),
        mesh=mesh,
        compiler_params=pltpu.CompilerParams(needs_layout_passes=False),
        scratch_types=[
            pltpu.VMEM((2, _CH), jnp.float32),
            pltpu.VMEM((_BINS * _L,), jnp.float32),
            pltpu.VMEM((_BINS * _L,), jnp.float32),
            pltpu.VMEM((_NS, _BINS), jnp.float32),
            pltpu.VMEM((_BINS,), jnp.float32),
            pltpu.VMEM((_L,), jnp.float32),
            pltpu.VMEM((_BINS,), jnp.float32),
            pltpu.VMEM((_BINS,), jnp.float32),
            pltpu.SemaphoreType.DMA,
            pltpu.SemaphoreType.DMA,
            pltpu.VMEM_SHARED((_NS, _BINS * _L), jnp.float32),
            pltpu.VMEM_SHARED((_NS, _BINS * _L), jnp.float32),
            pltpu.VMEM_SHARED((_BINS,), jnp.float32),
            pltpu.VMEM_SHARED((_BINS,), jnp.float32),
        ],
    )
    return f(loss)


def kernel(predict, target, is_average):
    loss = _bce_loss_channel_major(predict, target).reshape(_C, _N)
    sums = _sc_topk_sums(loss)
    total = (sums[0, 0] + sums[1, 0]) / (_KK * _C)
    return jnp.where(is_average, total, total * _B)
